# baseline (speedup 1.0000x reference)
import functools

import jax
import jax.numpy as jnp
from jax import lax
from jax.experimental import pallas as pl
from jax.experimental.pallas import tpu as pltpu
from jax.experimental.pallas import tpu_sc as plsc

N = 10000
D = 128
R = 32
E = 160000
E_PAD = 163840
BN_ = 1000
BE = 1024
CH = 128
CH0 = 64
NT = 32
NRC = 78
F32 = jnp.float32


def _mesh():
    return plsc.VectorSubcoreMesh(core_axis_name="c", subcore_axis_name="s")


def _sds(shape):
    return jax.ShapeDtypeStruct(shape, F32)


def _p1_body(x_ref, e_ref, w_ref, b_ref, o_ref):
    xt = x_ref[0, 0]
    oh = (xt[:, None] == lax.broadcasted_iota(jnp.int32, (BN_, D), 1)).astype(F32)
    emb = jnp.dot(oh, e_ref[...], preferred_element_type=F32)
    o_ref[...] = jnp.maximum(jnp.dot(emb, w_ref[...], preferred_element_type=F32)
                             + b_ref[0:1], 0.0)


def _p1(x3, emb_pad, pre_W, pre_b8):
    return pl.pallas_call(
        _p1_body,
        grid=(N // BN_,),
        in_specs=[pl.BlockSpec((1, 1, BN_), lambda j: (j, 0, 0)),
                  pl.BlockSpec((D, D), lambda j: (0, 0)),
                  pl.BlockSpec((D, D), lambda j: (0, 0)),
                  pl.BlockSpec((8, D), lambda j: (0, 0))],
        out_specs=pl.BlockSpec((BN_, D), lambda j: (j, 0)),
        out_shape=_sds((N, D)),
    )(x3, emb_pad, pre_W, pre_b8)


def _mk_gather(width, ch):
    cpt = E_PAD // ch // NT

    @functools.partial(
        pl.kernel, mesh=_mesh(),
        out_type=[_sds((E_PAD, width)), _sds((E_PAD, width))],
        scratch_types=[pltpu.VMEM((ch,), jnp.int32), pltpu.VMEM((ch,), jnp.int32),
                       pltpu.VMEM((ch, width), F32), pltpu.VMEM((ch, width), F32),
                       pltpu.SemaphoreType.DMA, pltpu.SemaphoreType.DMA],
    )
    def k(tab_hbm, dst_hbm, src_hbm, gd_hbm, gs_hbm,
          ixd, ixs, rd, rs, sd, ss):
        wid = lax.axis_index("c") * 16 + lax.axis_index("s")

        def body(t, carry):
            base = (wid + NT * t) * ch
            pltpu.sync_copy(dst_hbm.at[pl.ds(base, ch)], ixd)
            pltpu.sync_copy(src_hbm.at[pl.ds(base, ch)], ixs)
            c0 = pltpu.async_copy(tab_hbm.at[ixd], rd, sd)
            c1 = pltpu.async_copy(tab_hbm.at[ixs], rs, ss)
            c0.wait(); c1.wait()
            pltpu.sync_copy(rd, gd_hbm.at[pl.ds(base, ch)])
            pltpu.sync_copy(rs, gs_hbm.at[pl.ds(base, ch)])
            return carry

        lax.fori_loop(0, cpt, body, 0)

    return k


def _sc_gather0(t0, dstv, srcv):
    return _mk_gather(2 * D, CH)(t0, dstv, srcv)


def _sc_gather(h, dstv, srcv):
    return _mk_gather(D, CH)(h, dstv, srcv)


def _edge_math(gd, gs, rbf, wd_ref, ws_ref, wr_ref, bfs_ref):
    z = (jnp.dot(gd, wd_ref[...], preferred_element_type=F32)
         + jnp.dot(gs, ws_ref[...], preferred_element_type=F32)
         + jnp.dot(rbf, wr_ref[...], preferred_element_type=F32)
         + bfs_ref[0:1])
    return jax.nn.sigmoid(z[:, :D]) * jax.nn.softplus(z[:, D:])


def _tcb0_body(gd_ref, gs_ref, mn_ref, bt_ref,
               wd_ref, ws_ref, wr_ref, bfs_ref, m_ref, rbf_ref):
    gd = gd_ref[:, :D]
    gs = gs_ref[:, :D]
    vec = gs_ref[:, D:D + 16] - gd_ref[:, D:D + 16]
    d = jnp.sqrt(jnp.sum(vec * vec, axis=1, keepdims=True) + 1e-12)
    cut = jnp.where(d < 5.0, 0.5 * (jnp.cos(d * (jnp.pi / 5.0)) + 1.0), 0.0)
    rbf = cut * jnp.exp(-bt_ref[0:1] * (jnp.exp(-d) - mn_ref[0:1]) ** 2)
    rbf_ref[...] = rbf
    m_ref[...] = _edge_math(gd, gs, rbf, wd_ref, ws_ref, wr_ref, bfs_ref)


def _tcb0(gd, gs, means8, betas8, WD, WS, WR, bfs8):
    full = lambda s: pl.BlockSpec(s, lambda j: tuple(0 for _ in s))
    return pl.pallas_call(
        _tcb0_body,
        grid=(E_PAD // BE,),
        in_specs=[pl.BlockSpec((BE, 2 * D), lambda j: (j, 0)),
                  pl.BlockSpec((BE, 2 * D), lambda j: (j, 0)),
                  full((8, R)), full((8, R)),
                  full((D, 2 * D)), full((D, 2 * D)), full((R, 2 * D)),
                  full((8, 2 * D))],
        out_specs=[pl.BlockSpec((BE, D), lambda j: (j, 0)),
                   pl.BlockSpec((BE, R), lambda j: (j, 0))],
        out_shape=[_sds((E_PAD, D)), _sds((E_PAD, R))],
    )(gd, gs, means8, betas8, WD, WS, WR, bfs8)


def _tcb_body(gd_ref, gs_ref, rbf_ref, wd_ref, ws_ref, wr_ref, bfs_ref, m_ref):
    m_ref[...] = _edge_math(gd_ref[...], gs_ref[...], rbf_ref[...],
                            wd_ref, ws_ref, wr_ref, bfs_ref)


def _tcb(gd, gs, rbf, WD, WS, WR, bfs8):
    full = lambda s: pl.BlockSpec(s, lambda j: tuple(0 for _ in s))
    return pl.pallas_call(
        _tcb_body,
        grid=(E_PAD // BE,),
        in_specs=[pl.BlockSpec((BE, D), lambda j: (j, 0)),
                  pl.BlockSpec((BE, D), lambda j: (j, 0)),
                  pl.BlockSpec((BE, R), lambda j: (j, 0)),
                  full((D, 2 * D)), full((D, 2 * D)), full((R, 2 * D)),
                  full((8, 2 * D))],
        out_specs=pl.BlockSpec((BE, D), lambda j: (j, 0)),
        out_shape=_sds((E_PAD, D)),
    )(gd, gs, rbf, WD, WS, WR, bfs8)


def _zero_acc(s, zbuf, acc):
    for t in range(5):
        ci = s + 16 * t

        @pl.when(ci < NRC)
        def _():
            pltpu.sync_copy(zbuf, acc.at[pl.ds(ci * CH, CH)])

        @pl.when(ci == NRC)
        def _():
            pltpu.sync_copy(zbuf.at[pl.ds(0, 16)], acc.at[pl.ds(NRC * CH, 16)])


def _write_out(c, s, acc, buf, out_hbm):
    for t in range(5):
        ci = s + 16 * t

        @pl.when(ci < NRC)
        def _():
            pltpu.sync_copy(acc.at[pl.ds(ci * CH, CH)], buf)
            pltpu.sync_copy(buf, out_hbm.at[pl.ds(c * N + ci * CH, CH)])

        @pl.when(ci == NRC)
        def _():
            pltpu.sync_copy(acc.at[pl.ds(NRC * CH, 16)], buf.at[pl.ds(0, 16)])
            pltpu.sync_copy(buf.at[pl.ds(0, 16)],
                            out_hbm.at[pl.ds(c * N + NRC * CH, 16)])


_CPT = E_PAD // CH // NT
_KSC = _CPT // 2


def _sc_scatter(m, dstv, z128):
    @functools.partial(
        pl.kernel, mesh=_mesh(),
        out_type=_sds((2 * N, D)),
        scratch_types=[pltpu.VMEM((CH, D), F32), pltpu.VMEM((CH, D), F32),
                       pltpu.VMEM((CH,), jnp.int32), pltpu.VMEM((CH,), jnp.int32),
                       pltpu.VMEM_SHARED((N + 16, D), F32),
                       pltpu.SemaphoreType.DMA, pltpu.SemaphoreType.DMA,
                       pltpu.SemaphoreType.DMA, pltpu.SemaphoreType.DMA],
    )
    def k(m_hbm, dst_hbm, z_hbm, pout_hbm, mb0, mb1, ix0, ix1, acc,
          sm0, sm1, si0, si1):
        c = lax.axis_index("c")
        s = lax.axis_index("s")
        wid = c * 16 + s
        pltpu.sync_copy(z_hbm, mb0)
        _zero_acc(s, mb0, acc)
        plsc.subcore_barrier()

        def base(t):
            return (wid + NT * t) * CH

        pltpu.async_copy(dst_hbm.at[pl.ds(base(0), CH)], ix0, si0)
        pltpu.async_copy(m_hbm.at[pl.ds(base(0), CH)], mb0, sm0)

        def body(kk, carry):
            t0 = 2 * kk
            b0 = base(t0)
            b1 = base(t0 + 1)
            pltpu.async_copy(dst_hbm.at[pl.ds(b1, CH)], ix1, si1)
            pltpu.async_copy(m_hbm.at[pl.ds(b1, CH)], mb1, sm1)
            pltpu.make_async_copy(dst_hbm.at[pl.ds(b0, CH)], ix0, si0).wait()
            pltpu.make_async_copy(m_hbm.at[pl.ds(b0, CH)], mb0, sm0).wait()
            pltpu.sync_copy(mb0, acc.at[ix0], add=True)

            @pl.when(kk < _KSC - 1)
            def _():
                b2 = base(t0 + 2)
                pltpu.async_copy(dst_hbm.at[pl.ds(b2, CH)], ix0, si0)
                pltpu.async_copy(m_hbm.at[pl.ds(b2, CH)], mb0, sm0)

            pltpu.make_async_copy(dst_hbm.at[pl.ds(b1, CH)], ix1, si1).wait()
            pltpu.make_async_copy(m_hbm.at[pl.ds(b1, CH)], mb1, sm1).wait()
            pltpu.sync_copy(mb1, acc.at[ix1], add=True)
            return carry

        lax.fori_loop(0, _KSC, body, 0)
        plsc.subcore_barrier()
        _write_out(c, s, acc, mb0, pout_hbm)

    return k(m, dstv, z128)


def _sc_scatter_cnt(dstv, z128, o128):
    @functools.partial(
        pl.kernel, mesh=_mesh(),
        out_type=_sds((2 * N, D)),
        scratch_types=[pltpu.VMEM((CH, D), F32), pltpu.VMEM((CH, D), F32),
                       pltpu.VMEM((CH,), jnp.int32), pltpu.VMEM((CH,), jnp.int32),
                       pltpu.VMEM_SHARED((N + 16, D), F32),
                       pltpu.SemaphoreType.DMA, pltpu.SemaphoreType.DMA],
    )
    def k(dst_hbm, z_hbm, o_hbm, cout_hbm, zbuf, obuf, ix0, ix1, cacc, si0, si1):
        c = lax.axis_index("c")
        s = lax.axis_index("s")
        wid = c * 16 + s
        pltpu.sync_copy(z_hbm, zbuf)
        pltpu.sync_copy(o_hbm, obuf)
        _zero_acc(s, zbuf, cacc)
        plsc.subcore_barrier()

        def base(t):
            return (wid + NT * t) * CH

        pltpu.async_copy(dst_hbm.at[pl.ds(base(0), CH)], ix0, si0)

        def body(kk, carry):
            t0 = 2 * kk
            b0 = base(t0)
            b1 = base(t0 + 1)
            pltpu.async_copy(dst_hbm.at[pl.ds(b1, CH)], ix1, si1)
            pltpu.make_async_copy(dst_hbm.at[pl.ds(b0, CH)], ix0, si0).wait()
            pltpu.sync_copy(obuf, cacc.at[ix0], add=True)

            @pl.when(kk < _KSC - 1)
            def _():
                b2 = base(t0 + 2)
                pltpu.async_copy(dst_hbm.at[pl.ds(b2, CH)], ix0, si0)

            pltpu.make_async_copy(dst_hbm.at[pl.ds(b1, CH)], ix1, si1).wait()
            pltpu.sync_copy(obuf, cacc.at[ix1], add=True)
            return carry

        lax.fori_loop(0, _KSC, body, 0)
        plsc.subcore_barrier()
        _write_out(c, s, cacc, zbuf, cout_hbm)

    return k(dstv, z128, o128)


def _d1_0_body(h_ref, pa_ref, pb_ref, ca_ref, cb_ref, t_ref, su_ref, ic_ref):
    cnt = ca_ref[:, 0:1] + cb_ref[:, 0:1]
    ic = 1.0 / jnp.maximum(cnt, 1.0)
    ic_ref[...] = jnp.broadcast_to(ic, (BN_, 16))
    t = h_ref[...] + (pa_ref[...] + pb_ref[...]) * ic
    t_ref[...] = t
    s1 = jnp.sum(t, axis=0, keepdims=True)
    s2 = jnp.sum(t * t, axis=0, keepdims=True)
    su_ref[...] = jnp.concatenate([s1, s2, jnp.zeros((6, D), F32)], axis=0)[None]


def _d1_0(h, pout, cout):
    nb = N // BN_
    return pl.pallas_call(
        _d1_0_body,
        grid=(nb,),
        in_specs=[pl.BlockSpec((BN_, D), lambda j: (j, 0)),
                  pl.BlockSpec((BN_, D), lambda j: (j, 0)),
                  pl.BlockSpec((BN_, D), lambda j: (j + nb, 0)),
                  pl.BlockSpec((BN_, D), lambda j: (j, 0)),
                  pl.BlockSpec((BN_, D), lambda j: (j + nb, 0))],
        out_specs=[pl.BlockSpec((BN_, D), lambda j: (j, 0)),
                   pl.BlockSpec((1, 8, D), lambda j: (j, 0, 0)),
                   pl.BlockSpec((BN_, 16), lambda j: (j, 0))],
        out_shape=[_sds((N, D)), _sds((nb, 8, D)), _sds((N, 16))],
    )(h, pout, pout, cout, cout)


def _d1_body(h_ref, pa_ref, pb_ref, ic_ref, t_ref, su_ref):
    t = h_ref[...] + (pa_ref[...] + pb_ref[...]) * ic_ref[:, 0:1]
    t_ref[...] = t
    s1 = jnp.sum(t, axis=0, keepdims=True)
    s2 = jnp.sum(t * t, axis=0, keepdims=True)
    su_ref[...] = jnp.concatenate([s1, s2, jnp.zeros((6, D), F32)], axis=0)[None]


def _d1(h, pout, icnt):
    nb = N // BN_
    return pl.pallas_call(
        _d1_body,
        grid=(nb,),
        in_specs=[pl.BlockSpec((BN_, D), lambda j: (j, 0)),
                  pl.BlockSpec((BN_, D), lambda j: (j, 0)),
                  pl.BlockSpec((BN_, D), lambda j: (j + nb, 0)),
                  pl.BlockSpec((BN_, 16), lambda j: (j, 0))],
        out_specs=[pl.BlockSpec((BN_, D), lambda j: (j, 0)),
                   pl.BlockSpec((1, 8, D), lambda j: (j, 0, 0))],
        out_shape=[_sds((N, D)), _sds((nb, 8, D))],
    )(h, pout, pout, icnt)


def _d2_body(t_ref, su_ref, g_ref, b_ref, o_ref):
    su = su_ref[...]
    mu = jnp.sum(su[:, 0, :], axis=0) * (1.0 / N)
    ex2 = jnp.sum(su[:, 1, :], axis=0) * (1.0 / N)
    var = ex2 - mu * mu
    o_ref[...] = ((t_ref[...] - mu[None, :]) * lax.rsqrt(var + 1e-5)[None, :]
                  * g_ref[0:1] + b_ref[0:1])


def _d2(t, sums, g8, b8):
    nb = N // BN_
    return pl.pallas_call(
        _d2_body,
        grid=(nb,),
        in_specs=[pl.BlockSpec((BN_, D), lambda j: (j, 0)),
                  pl.BlockSpec((nb, 8, D), lambda j: (0, 0, 0)),
                  pl.BlockSpec((8, D), lambda j: (0, 0)),
                  pl.BlockSpec((8, D), lambda j: (0, 0))],
        out_specs=pl.BlockSpec((BN_, D), lambda j: (j, 0)),
        out_shape=_sds((N, D)),
    )(t, sums, g8, b8)


def kernel(x, pos, edge_index, emb_table, pre_W, pre_b, rbf_means, rbf_betas,
           Wf0, bf0, Ws0, bs0, gamma0, beta0,
           Wf1, bf1, Ws1, bs1, gamma1, beta1,
           Wf2, bf2, Ws2, bs2, gamma2, beta2):
    src = edge_index[0].astype(jnp.int32)
    dst = edge_index[1].astype(jnp.int32)
    pad_e = E_PAD - E
    zpad = jnp.zeros((pad_e,), jnp.int32)
    dst_g = jnp.concatenate([dst, zpad])
    src_g = jnp.concatenate([src, zpad])
    dst_s = jnp.concatenate([dst, jnp.full((pad_e,), N, jnp.int32)])
    x3 = x.astype(jnp.int32).reshape(N // BN_, 1, BN_)
    pos16 = jnp.pad(pos.astype(F32), ((0, 0), (0, 13)))
    emb_pad = jnp.pad(emb_table, ((0, D - emb_table.shape[0]), (0, 0)))
    bc8 = lambda v: jnp.broadcast_to(v[None, :], (8, v.shape[0]))
    z128 = jnp.zeros((CH, D), F32)
    o128 = jnp.ones((CH, D), F32)

    layers = []
    for (Wf, bf, Ws, bs, g, b) in ((Wf0, bf0, Ws0, bs0, gamma0, beta0),
                                   (Wf1, bf1, Ws1, bs1, gamma1, beta1),
                                   (Wf2, bf2, Ws2, bs2, gamma2, beta2)):
        WD = jnp.concatenate([Wf[:D], Ws[:D]], axis=1)
        WS = jnp.concatenate([Wf[D:2 * D], Ws[D:2 * D]], axis=1)
        WR = jnp.concatenate([Wf[2 * D:], Ws[2 * D:]], axis=1)
        bfs8 = bc8(jnp.concatenate([bf, bs]))
        layers.append((WD, WS, WR, bfs8, bc8(g), bc8(b)))

    h = _p1(x3, emb_pad, pre_W, bc8(pre_b))

    rbf = None
    icnt = None
    for i, (WD, WS, WR, bfs8, g8, b8) in enumerate(layers):
        if i == 0:
            t0 = jnp.concatenate([h, pos16, jnp.zeros((N, D - 16), F32)], axis=1)
            gd, gs = _sc_gather0(t0, dst_g, src_g)
            m, rbf = _tcb0(gd, gs, bc8(rbf_means), bc8(rbf_betas),
                           WD, WS, WR, bfs8)
            pout = _sc_scatter(m, dst_s, z128)
            cout = _sc_scatter_cnt(dst_s, z128, o128)
            t, sums, icnt = _d1_0(h, pout, cout)
        else:
            gd, gs = _sc_gather(h, dst_g, src_g)
            m = _tcb(gd, gs, rbf, WD, WS, WR, bfs8)
            pout = _sc_scatter(m, dst_s, z128)
            t, sums = _d1(h, pout, icnt)
        h = _d2(t, sums, g8, b8)
    return h


# speedup vs baseline: 1.0002x; 1.0002x over previous
import functools

import jax
import jax.numpy as jnp
from jax import lax
from jax.experimental import pallas as pl
from jax.experimental.pallas import tpu as pltpu
from jax.experimental.pallas import tpu_sc as plsc

N = 10000
D = 128
R = 32
E = 160000
E_PAD = 163840
BN_ = 1000
BE = 1024
CH = 128
CH0 = 64
NT = 32
NRC = 78
F32 = jnp.float32


def _mesh():
    return plsc.VectorSubcoreMesh(core_axis_name="c", subcore_axis_name="s")


def _sds(shape):
    return jax.ShapeDtypeStruct(shape, F32)


def _p1_body(x_ref, e_ref, w_ref, b_ref, o_ref):
    xt = x_ref[0, 0]
    oh = (xt[:, None] == lax.broadcasted_iota(jnp.int32, (BN_, D), 1)).astype(F32)
    emb = jnp.dot(oh, e_ref[...], preferred_element_type=F32)
    o_ref[...] = jnp.maximum(jnp.dot(emb, w_ref[...], preferred_element_type=F32)
                             + b_ref[0:1], 0.0)


def _p1(x3, emb_pad, pre_W, pre_b8):
    return pl.pallas_call(
        _p1_body,
        grid=(N // BN_,),
        in_specs=[pl.BlockSpec((1, 1, BN_), lambda j: (j, 0, 0)),
                  pl.BlockSpec((D, D), lambda j: (0, 0)),
                  pl.BlockSpec((D, D), lambda j: (0, 0)),
                  pl.BlockSpec((8, D), lambda j: (0, 0))],
        out_specs=pl.BlockSpec((BN_, D), lambda j: (j, 0)),
        out_shape=_sds((N, D)),
    )(x3, emb_pad, pre_W, pre_b8)


def _mk_gather(width, ch):
    cpt = E_PAD // ch // NT

    @functools.partial(
        pl.kernel, mesh=_mesh(),
        out_type=[_sds((E_PAD, width)), _sds((E_PAD, width))],
        scratch_types=[pltpu.VMEM((ch,), jnp.int32), pltpu.VMEM((ch,), jnp.int32),
                       pltpu.VMEM((ch, width), F32), pltpu.VMEM((ch, width), F32),
                       pltpu.SemaphoreType.DMA, pltpu.SemaphoreType.DMA],
    )
    def k(tab_hbm, dst_hbm, src_hbm, gd_hbm, gs_hbm,
          ixd, ixs, rd, rs, sd, ss):
        wid = lax.axis_index("c") * 16 + lax.axis_index("s")
        nc = jnp.minimum(wid * 0 + cpt, cpt)

        def body(t, carry):
            base = (wid + NT * t) * ch
            pltpu.sync_copy(dst_hbm.at[pl.ds(base, ch)], ixd)
            pltpu.sync_copy(src_hbm.at[pl.ds(base, ch)], ixs)
            c0 = pltpu.async_copy(tab_hbm.at[ixd], rd, sd)
            c1 = pltpu.async_copy(tab_hbm.at[ixs], rs, ss)
            c0.wait(); c1.wait()
            pltpu.sync_copy(rd, gd_hbm.at[pl.ds(base, ch)])
            pltpu.sync_copy(rs, gs_hbm.at[pl.ds(base, ch)])
            return carry

        lax.fori_loop(0, nc, body, 0)

    return k


def _sc_gather0(t0, dstv, srcv):
    return _mk_gather(2 * D, CH)(t0, dstv, srcv)


def _sc_gather(h, dstv, srcv):
    return _mk_gather(D, CH)(h, dstv, srcv)


def _edge_math(gd, gs, rbf, wd_ref, ws_ref, wr_ref, bfs_ref):
    z = (jnp.dot(gd, wd_ref[...], preferred_element_type=F32)
         + jnp.dot(gs, ws_ref[...], preferred_element_type=F32)
         + jnp.dot(rbf, wr_ref[...], preferred_element_type=F32)
         + bfs_ref[0:1])
    return jax.nn.sigmoid(z[:, :D]) * jax.nn.softplus(z[:, D:])


def _tcb0_body(gd_ref, gs_ref, mn_ref, bt_ref,
               wd_ref, ws_ref, wr_ref, bfs_ref, m_ref, rbf_ref):
    gd = gd_ref[:, :D]
    gs = gs_ref[:, :D]
    vec = gs_ref[:, D:D + 16] - gd_ref[:, D:D + 16]
    d = jnp.sqrt(jnp.sum(vec * vec, axis=1, keepdims=True) + 1e-12)
    cut = jnp.where(d < 5.0, 0.5 * (jnp.cos(d * (jnp.pi / 5.0)) + 1.0), 0.0)
    rbf = cut * jnp.exp(-bt_ref[0:1] * (jnp.exp(-d) - mn_ref[0:1]) ** 2)
    rbf_ref[...] = rbf
    m_ref[...] = _edge_math(gd, gs, rbf, wd_ref, ws_ref, wr_ref, bfs_ref)


def _tcb0(gd, gs, means8, betas8, WD, WS, WR, bfs8):
    full = lambda s: pl.BlockSpec(s, lambda j: tuple(0 for _ in s))
    return pl.pallas_call(
        _tcb0_body,
        grid=(E_PAD // BE,),
        in_specs=[pl.BlockSpec((BE, 2 * D), lambda j: (j, 0)),
                  pl.BlockSpec((BE, 2 * D), lambda j: (j, 0)),
                  full((8, R)), full((8, R)),
                  full((D, 2 * D)), full((D, 2 * D)), full((R, 2 * D)),
                  full((8, 2 * D))],
        out_specs=[pl.BlockSpec((BE, D), lambda j: (j, 0)),
                   pl.BlockSpec((BE, R), lambda j: (j, 0))],
        out_shape=[_sds((E_PAD, D)), _sds((E_PAD, R))],
    )(gd, gs, means8, betas8, WD, WS, WR, bfs8)


def _tcb_body(gd_ref, gs_ref, rbf_ref, wd_ref, ws_ref, wr_ref, bfs_ref, m_ref):
    m_ref[...] = _edge_math(gd_ref[...], gs_ref[...], rbf_ref[...],
                            wd_ref, ws_ref, wr_ref, bfs_ref)


def _tcb(gd, gs, rbf, WD, WS, WR, bfs8):
    full = lambda s: pl.BlockSpec(s, lambda j: tuple(0 for _ in s))
    return pl.pallas_call(
        _tcb_body,
        grid=(E_PAD // BE,),
        in_specs=[pl.BlockSpec((BE, D), lambda j: (j, 0)),
                  pl.BlockSpec((BE, D), lambda j: (j, 0)),
                  pl.BlockSpec((BE, R), lambda j: (j, 0)),
                  full((D, 2 * D)), full((D, 2 * D)), full((R, 2 * D)),
                  full((8, 2 * D))],
        out_specs=pl.BlockSpec((BE, D), lambda j: (j, 0)),
        out_shape=_sds((E_PAD, D)),
    )(gd, gs, rbf, WD, WS, WR, bfs8)


def _zero_acc(s, zbuf, acc):
    for t in range(5):
        ci = s + 16 * t

        @pl.when(ci < NRC)
        def _():
            pltpu.sync_copy(zbuf, acc.at[pl.ds(ci * CH, CH)])

        @pl.when(ci == NRC)
        def _():
            pltpu.sync_copy(zbuf.at[pl.ds(0, 16)], acc.at[pl.ds(NRC * CH, 16)])


def _write_out(c, s, acc, buf, out_hbm):
    for t in range(5):
        ci = s + 16 * t

        @pl.when(ci < NRC)
        def _():
            pltpu.sync_copy(acc.at[pl.ds(ci * CH, CH)], buf)
            pltpu.sync_copy(buf, out_hbm.at[pl.ds(c * N + ci * CH, CH)])

        @pl.when(ci == NRC)
        def _():
            pltpu.sync_copy(acc.at[pl.ds(NRC * CH, 16)], buf.at[pl.ds(0, 16)])
            pltpu.sync_copy(buf.at[pl.ds(0, 16)],
                            out_hbm.at[pl.ds(c * N + NRC * CH, 16)])


_CPT = E_PAD // CH // NT
_KSC = _CPT // 2


def _sc_scatter(m, dstv, z128):
    @functools.partial(
        pl.kernel, mesh=_mesh(),
        out_type=_sds((2 * N, D)),
        scratch_types=[pltpu.VMEM((CH, D), F32), pltpu.VMEM((CH, D), F32),
                       pltpu.VMEM((CH,), jnp.int32), pltpu.VMEM((CH,), jnp.int32),
                       pltpu.VMEM_SHARED((N + 16, D), F32),
                       pltpu.SemaphoreType.DMA, pltpu.SemaphoreType.DMA,
                       pltpu.SemaphoreType.DMA, pltpu.SemaphoreType.DMA],
    )
    def k(m_hbm, dst_hbm, z_hbm, pout_hbm, mb0, mb1, ix0, ix1, acc,
          sm0, sm1, si0, si1):
        c = lax.axis_index("c")
        s = lax.axis_index("s")
        wid = c * 16 + s
        pltpu.sync_copy(z_hbm, mb0)
        _zero_acc(s, mb0, acc)
        plsc.subcore_barrier()

        def base(t):
            return (wid + NT * t) * CH

        pltpu.async_copy(dst_hbm.at[pl.ds(base(0), CH)], ix0, si0)
        pltpu.async_copy(m_hbm.at[pl.ds(base(0), CH)], mb0, sm0)

        def body(kk, carry):
            t0 = 2 * kk
            b0 = base(t0)
            b1 = base(t0 + 1)
            pltpu.async_copy(dst_hbm.at[pl.ds(b1, CH)], ix1, si1)
            pltpu.async_copy(m_hbm.at[pl.ds(b1, CH)], mb1, sm1)
            pltpu.make_async_copy(dst_hbm.at[pl.ds(b0, CH)], ix0, si0).wait()
            pltpu.make_async_copy(m_hbm.at[pl.ds(b0, CH)], mb0, sm0).wait()
            pltpu.sync_copy(mb0, acc.at[ix0], add=True)

            @pl.when(kk < _KSC - 1)
            def _():
                b2 = base(t0 + 2)
                pltpu.async_copy(dst_hbm.at[pl.ds(b2, CH)], ix0, si0)
                pltpu.async_copy(m_hbm.at[pl.ds(b2, CH)], mb0, sm0)

            pltpu.make_async_copy(dst_hbm.at[pl.ds(b1, CH)], ix1, si1).wait()
            pltpu.make_async_copy(m_hbm.at[pl.ds(b1, CH)], mb1, sm1).wait()
            pltpu.sync_copy(mb1, acc.at[ix1], add=True)
            return carry

        lax.fori_loop(0, _KSC, body, 0)
        plsc.subcore_barrier()
        _write_out(c, s, acc, mb0, pout_hbm)

    return k(m, dstv, z128)


def _sc_scatter_cnt(dstv, z128, o128):
    @functools.partial(
        pl.kernel, mesh=_mesh(),
        out_type=_sds((2 * N, D)),
        scratch_types=[pltpu.VMEM((CH, D), F32), pltpu.VMEM((CH, D), F32),
                       pltpu.VMEM((CH,), jnp.int32), pltpu.VMEM((CH,), jnp.int32),
                       pltpu.VMEM_SHARED((N + 16, D), F32),
                       pltpu.SemaphoreType.DMA, pltpu.SemaphoreType.DMA],
    )
    def k(dst_hbm, z_hbm, o_hbm, cout_hbm, zbuf, obuf, ix0, ix1, cacc, si0, si1):
        c = lax.axis_index("c")
        s = lax.axis_index("s")
        wid = c * 16 + s
        pltpu.sync_copy(z_hbm, zbuf)
        pltpu.sync_copy(o_hbm, obuf)
        _zero_acc(s, zbuf, cacc)
        plsc.subcore_barrier()

        def base(t):
            return (wid + NT * t) * CH

        pltpu.async_copy(dst_hbm.at[pl.ds(base(0), CH)], ix0, si0)

        def body(kk, carry):
            t0 = 2 * kk
            b0 = base(t0)
            b1 = base(t0 + 1)
            pltpu.async_copy(dst_hbm.at[pl.ds(b1, CH)], ix1, si1)
            pltpu.make_async_copy(dst_hbm.at[pl.ds(b0, CH)], ix0, si0).wait()
            pltpu.sync_copy(obuf, cacc.at[ix0], add=True)

            @pl.when(kk < _KSC - 1)
            def _():
                b2 = base(t0 + 2)
                pltpu.async_copy(dst_hbm.at[pl.ds(b2, CH)], ix0, si0)

            pltpu.make_async_copy(dst_hbm.at[pl.ds(b1, CH)], ix1, si1).wait()
            pltpu.sync_copy(obuf, cacc.at[ix1], add=True)
            return carry

        lax.fori_loop(0, _KSC, body, 0)
        plsc.subcore_barrier()
        _write_out(c, s, cacc, zbuf, cout_hbm)

    return k(dstv, z128, o128)


def _d1_0_body(h_ref, pa_ref, pb_ref, ca_ref, cb_ref, t_ref, su_ref, ic_ref):
    cnt = ca_ref[:, 0:1] + cb_ref[:, 0:1]
    ic = 1.0 / jnp.maximum(cnt, 1.0)
    ic_ref[...] = jnp.broadcast_to(ic, (BN_, 16))
    t = h_ref[...] + (pa_ref[...] + pb_ref[...]) * ic
    t_ref[...] = t
    s1 = jnp.sum(t, axis=0, keepdims=True)
    s2 = jnp.sum(t * t, axis=0, keepdims=True)
    su_ref[...] = jnp.concatenate([s1, s2, jnp.zeros((6, D), F32)], axis=0)[None]


def _d1_0(h, pout, cout):
    nb = N // BN_
    return pl.pallas_call(
        _d1_0_body,
        grid=(nb,),
        in_specs=[pl.BlockSpec((BN_, D), lambda j: (j, 0)),
                  pl.BlockSpec((BN_, D), lambda j: (j, 0)),
                  pl.BlockSpec((BN_, D), lambda j: (j + nb, 0)),
                  pl.BlockSpec((BN_, D), lambda j: (j, 0)),
                  pl.BlockSpec((BN_, D), lambda j: (j + nb, 0))],
        out_specs=[pl.BlockSpec((BN_, D), lambda j: (j, 0)),
                   pl.BlockSpec((1, 8, D), lambda j: (j, 0, 0)),
                   pl.BlockSpec((BN_, 16), lambda j: (j, 0))],
        out_shape=[_sds((N, D)), _sds((nb, 8, D)), _sds((N, 16))],
    )(h, pout, pout, cout, cout)


def _d1_body(h_ref, pa_ref, pb_ref, ic_ref, t_ref, su_ref):
    t = h_ref[...] + (pa_ref[...] + pb_ref[...]) * ic_ref[:, 0:1]
    t_ref[...] = t
    s1 = jnp.sum(t, axis=0, keepdims=True)
    s2 = jnp.sum(t * t, axis=0, keepdims=True)
    su_ref[...] = jnp.concatenate([s1, s2, jnp.zeros((6, D), F32)], axis=0)[None]


def _d1(h, pout, icnt):
    nb = N // BN_
    return pl.pallas_call(
        _d1_body,
        grid=(nb,),
        in_specs=[pl.BlockSpec((BN_, D), lambda j: (j, 0)),
                  pl.BlockSpec((BN_, D), lambda j: (j, 0)),
                  pl.BlockSpec((BN_, D), lambda j: (j + nb, 0)),
                  pl.BlockSpec((BN_, 16), lambda j: (j, 0))],
        out_specs=[pl.BlockSpec((BN_, D), lambda j: (j, 0)),
                   pl.BlockSpec((1, 8, D), lambda j: (j, 0, 0))],
        out_shape=[_sds((N, D)), _sds((nb, 8, D))],
    )(h, pout, pout, icnt)


def _d2_body(t_ref, su_ref, g_ref, b_ref, o_ref):
    su = su_ref[...]
    mu = jnp.sum(su[:, 0, :], axis=0) * (1.0 / N)
    ex2 = jnp.sum(su[:, 1, :], axis=0) * (1.0 / N)
    var = ex2 - mu * mu
    o_ref[...] = ((t_ref[...] - mu[None, :]) * lax.rsqrt(var + 1e-5)[None, :]
                  * g_ref[0:1] + b_ref[0:1])


def _d2(t, sums, g8, b8):
    nb = N // BN_
    return pl.pallas_call(
        _d2_body,
        grid=(nb,),
        in_specs=[pl.BlockSpec((BN_, D), lambda j: (j, 0)),
                  pl.BlockSpec((nb, 8, D), lambda j: (0, 0, 0)),
                  pl.BlockSpec((8, D), lambda j: (0, 0)),
                  pl.BlockSpec((8, D), lambda j: (0, 0))],
        out_specs=pl.BlockSpec((BN_, D), lambda j: (j, 0)),
        out_shape=_sds((N, D)),
    )(t, sums, g8, b8)


def kernel(x, pos, edge_index, emb_table, pre_W, pre_b, rbf_means, rbf_betas,
           Wf0, bf0, Ws0, bs0, gamma0, beta0,
           Wf1, bf1, Ws1, bs1, gamma1, beta1,
           Wf2, bf2, Ws2, bs2, gamma2, beta2):
    src = edge_index[0].astype(jnp.int32)
    dst = edge_index[1].astype(jnp.int32)
    pad_e = E_PAD - E
    zpad = jnp.zeros((pad_e,), jnp.int32)
    dst_g = jnp.concatenate([dst, zpad])
    src_g = jnp.concatenate([src, zpad])
    dst_s = jnp.concatenate([dst, jnp.full((pad_e,), N, jnp.int32)])
    x3 = x.astype(jnp.int32).reshape(N // BN_, 1, BN_)
    pos16 = jnp.pad(pos.astype(F32), ((0, 0), (0, 13)))
    emb_pad = jnp.pad(emb_table, ((0, D - emb_table.shape[0]), (0, 0)))
    bc8 = lambda v: jnp.broadcast_to(v[None, :], (8, v.shape[0]))
    z128 = jnp.zeros((CH, D), F32)
    o128 = jnp.ones((CH, D), F32)

    layers = []
    for (Wf, bf, Ws, bs, g, b) in ((Wf0, bf0, Ws0, bs0, gamma0, beta0),
                                   (Wf1, bf1, Ws1, bs1, gamma1, beta1),
                                   (Wf2, bf2, Ws2, bs2, gamma2, beta2)):
        WD = jnp.concatenate([Wf[:D], Ws[:D]], axis=1)
        WS = jnp.concatenate([Wf[D:2 * D], Ws[D:2 * D]], axis=1)
        WR = jnp.concatenate([Wf[2 * D:], Ws[2 * D:]], axis=1)
        bfs8 = bc8(jnp.concatenate([bf, bs]))
        layers.append((WD, WS, WR, bfs8, bc8(g), bc8(b)))

    h = _p1(x3, emb_pad, pre_W, bc8(pre_b))

    rbf = None
    icnt = None
    for i, (WD, WS, WR, bfs8, g8, b8) in enumerate(layers):
        if i == 0:
            t0 = jnp.concatenate([h, pos16, jnp.zeros((N, D - 16), F32)], axis=1)
            gd, gs = _sc_gather0(t0, dst_g, src_g)
            m, rbf = _tcb0(gd, gs, bc8(rbf_means), bc8(rbf_betas),
                           WD, WS, WR, bfs8)
            pout = _sc_scatter(m, dst_s, z128)
            cout = _sc_scatter_cnt(dst_s, z128, o128)
            t, sums, icnt = _d1_0(h, pout, cout)
        else:
            gd, gs = _sc_gather(h, dst_g, src_g)
            m = _tcb(gd, gs, rbf, WD, WS, WR, bfs8)
            pout = _sc_scatter(m, dst_s, z128)
            t, sums = _d1(h, pout, icnt)
        h = _d2(t, sums, g8, b8)
    return h


# speedup vs baseline: 1.3905x; 1.3902x over previous
import functools

import jax
import jax.numpy as jnp
from jax import lax
from jax.experimental import pallas as pl
from jax.experimental.pallas import tpu as pltpu
from jax.experimental.pallas import tpu_sc as plsc

N = 10000
D = 128
R = 32
E = 160000
BN_ = 1000
BE = 1000
CH = 128
CH0 = 64
NT = 32
NRC = 78
F32 = jnp.float32


def _mesh():
    return plsc.VectorSubcoreMesh(core_axis_name="c", subcore_axis_name="s")


def _sds(shape):
    return jax.ShapeDtypeStruct(shape, F32)


def _p1_body(x_ref, e_ref, w_ref, b_ref, o_ref):
    xt = x_ref[0, 0]
    oh = (xt[:, None] == lax.broadcasted_iota(jnp.int32, (BN_, D), 1)).astype(F32)
    emb = jnp.dot(oh, e_ref[...], preferred_element_type=F32)
    o_ref[...] = jnp.maximum(jnp.dot(emb, w_ref[...], preferred_element_type=F32)
                             + b_ref[0:1], 0.0)


def _p1(x3, emb_pad, pre_W, pre_b8):
    return pl.pallas_call(
        _p1_body,
        grid=(N // BN_,),
        in_specs=[pl.BlockSpec((1, 1, BN_), lambda j: (j, 0, 0)),
                  pl.BlockSpec((D, D), lambda j: (0, 0)),
                  pl.BlockSpec((D, D), lambda j: (0, 0)),
                  pl.BlockSpec((8, D), lambda j: (0, 0))],
        out_specs=pl.BlockSpec((BN_, D), lambda j: (j, 0)),
        out_shape=_sds((N, D)),
    )(x3, emb_pad, pre_W, pre_b8)


def _mk_gather(width, ch):
    nch = E // ch

    @functools.partial(
        pl.kernel, mesh=_mesh(),
        out_type=[_sds((E, width)), _sds((E, width))],
        scratch_types=[pltpu.VMEM((ch,), jnp.int32), pltpu.VMEM((ch,), jnp.int32),
                       pltpu.VMEM((ch, width), F32), pltpu.VMEM((ch, width), F32),
                       pltpu.SemaphoreType.DMA, pltpu.SemaphoreType.DMA],
    )
    def k(tab_hbm, dst_hbm, src_hbm, gd_hbm, gs_hbm,
          ixd, ixs, rd, rs, sd, ss):
        wid = lax.axis_index("c") * 16 + lax.axis_index("s")
        nc = jnp.where(wid < nch % NT, nch // NT + 1, nch // NT)

        def body(t, carry):
            base = (wid + NT * t) * ch
            pltpu.sync_copy(dst_hbm.at[pl.ds(base, ch)], ixd)
            pltpu.sync_copy(src_hbm.at[pl.ds(base, ch)], ixs)
            c0 = pltpu.async_copy(tab_hbm.at[ixd], rd, sd)
            c1 = pltpu.async_copy(tab_hbm.at[ixs], rs, ss)
            c0.wait(); c1.wait()
            pltpu.sync_copy(rd, gd_hbm.at[pl.ds(base, ch)])
            pltpu.sync_copy(rs, gs_hbm.at[pl.ds(base, ch)])
            return carry

        lax.fori_loop(0, nc, body, 0)

    return k


def _sc_gather0(t0, dstv, srcv):
    return _mk_gather(2 * D, CH)(t0, dstv, srcv)


def _sc_gather(h, dstv, srcv):
    return _mk_gather(D, CH)(h, dstv, srcv)


def _edge_math(gd, gs, rbf, wd_ref, ws_ref, wr_ref, bfs_ref):
    z = (jnp.dot(gd, wd_ref[...], preferred_element_type=F32)
         + jnp.dot(gs, ws_ref[...], preferred_element_type=F32)
         + jnp.dot(rbf, wr_ref[...], preferred_element_type=F32)
         + bfs_ref[0:1])
    return jax.nn.sigmoid(z[:, :D]) * jax.nn.softplus(z[:, D:])


def _tcb0_body(gd_ref, gs_ref, mn_ref, bt_ref,
               wd_ref, ws_ref, wr_ref, bfs_ref, m_ref, rbf_ref):
    gd = gd_ref[:, :D]
    gs = gs_ref[:, :D]
    vec = gs_ref[:, D:D + 16] - gd_ref[:, D:D + 16]
    d = jnp.sqrt(jnp.sum(vec * vec, axis=1, keepdims=True) + 1e-12)
    cut = jnp.where(d < 5.0, 0.5 * (jnp.cos(d * (jnp.pi / 5.0)) + 1.0), 0.0)
    rbf = cut * jnp.exp(-bt_ref[0:1] * (jnp.exp(-d) - mn_ref[0:1]) ** 2)
    rbf_ref[...] = rbf
    m_ref[...] = _edge_math(gd, gs, rbf, wd_ref, ws_ref, wr_ref, bfs_ref)


def _tcb0(gd, gs, means8, betas8, WD, WS, WR, bfs8):
    full = lambda s: pl.BlockSpec(s, lambda j: tuple(0 for _ in s))
    return pl.pallas_call(
        _tcb0_body,
        grid=(E // BE,),
        in_specs=[pl.BlockSpec((BE, 2 * D), lambda j: (j, 0)),
                  pl.BlockSpec((BE, 2 * D), lambda j: (j, 0)),
                  full((8, R)), full((8, R)),
                  full((D, 2 * D)), full((D, 2 * D)), full((R, 2 * D)),
                  full((8, 2 * D))],
        out_specs=[pl.BlockSpec((BE, D), lambda j: (j, 0)),
                   pl.BlockSpec((BE, R), lambda j: (j, 0))],
        out_shape=[_sds((E, D)), _sds((E, R))],
    )(gd, gs, means8, betas8, WD, WS, WR, bfs8)


def _tcb_body(gd_ref, gs_ref, rbf_ref, wd_ref, ws_ref, wr_ref, bfs_ref, m_ref):
    m_ref[...] = _edge_math(gd_ref[...], gs_ref[...], rbf_ref[...],
                            wd_ref, ws_ref, wr_ref, bfs_ref)


def _tcb(gd, gs, rbf, WD, WS, WR, bfs8):
    full = lambda s: pl.BlockSpec(s, lambda j: tuple(0 for _ in s))
    return pl.pallas_call(
        _tcb_body,
        grid=(E // BE,),
        in_specs=[pl.BlockSpec((BE, D), lambda j: (j, 0)),
                  pl.BlockSpec((BE, D), lambda j: (j, 0)),
                  pl.BlockSpec((BE, R), lambda j: (j, 0)),
                  full((D, 2 * D)), full((D, 2 * D)), full((R, 2 * D)),
                  full((8, 2 * D))],
        out_specs=pl.BlockSpec((BE, D), lambda j: (j, 0)),
        out_shape=_sds((E, D)),
    )(gd, gs, rbf, WD, WS, WR, bfs8)


def _zero_acc(s, zbuf, acc):
    for t in range(5):
        ci = s + 16 * t

        @pl.when(ci < NRC)
        def _():
            pltpu.sync_copy(zbuf, acc.at[pl.ds(ci * CH, CH)])

        @pl.when(ci == NRC)
        def _():
            pltpu.sync_copy(zbuf.at[pl.ds(0, 16)], acc.at[pl.ds(NRC * CH, 16)])


def _write_out(c, s, acc, buf, out_hbm):
    for t in range(5):
        ci = s + 16 * t

        @pl.when(ci < NRC)
        def _():
            pltpu.sync_copy(acc.at[pl.ds(ci * CH, CH)], buf)
            pltpu.sync_copy(buf, out_hbm.at[pl.ds(c * N + ci * CH, CH)])

        @pl.when(ci == NRC)
        def _():
            pltpu.sync_copy(acc.at[pl.ds(NRC * CH, 16)], buf.at[pl.ds(0, 16)])
            pltpu.sync_copy(buf.at[pl.ds(0, 16)],
                            out_hbm.at[pl.ds(c * N + NRC * CH, 16)])


def _sc_scatter(m, dstv, z128):
    @functools.partial(
        pl.kernel, mesh=_mesh(),
        out_type=_sds((2 * N, D)),
        scratch_types=[pltpu.VMEM((CH, D), F32), pltpu.VMEM((CH,), jnp.int32),
                       pltpu.VMEM_SHARED((N, D), F32)],
    )
    def k(m_hbm, dst_hbm, z_hbm, pout_hbm, mbuf, idx, acc):
        c = lax.axis_index("c")
        s = lax.axis_index("s")
        wid = c * 16 + s
        pltpu.sync_copy(z_hbm, mbuf)
        _zero_acc(s, mbuf, acc)
        plsc.subcore_barrier()
        nch = E // CH
        nc = jnp.where(wid < nch % NT, nch // NT + 1, nch // NT)

        def body(t, carry):
            base = (wid + NT * t) * CH
            pltpu.sync_copy(dst_hbm.at[pl.ds(base, CH)], idx)
            pltpu.sync_copy(m_hbm.at[pl.ds(base, CH)], mbuf)
            pltpu.sync_copy(mbuf, acc.at[idx], add=True)
            return carry

        lax.fori_loop(0, nc, body, 0)
        plsc.subcore_barrier()
        _write_out(c, s, acc, mbuf, pout_hbm)

    return k(m, dstv, z128)


def _sc_scatter_cnt(dstv, z128, o128):
    @functools.partial(
        pl.kernel, mesh=_mesh(),
        out_type=_sds((2 * N, D)),
        scratch_types=[pltpu.VMEM((CH, D), F32), pltpu.VMEM((CH, D), F32),
                       pltpu.VMEM((CH,), jnp.int32),
                       pltpu.VMEM_SHARED((N, D), F32)],
    )
    def k(dst_hbm, z_hbm, o_hbm, cout_hbm, zbuf, obuf, idx, cacc):
        c = lax.axis_index("c")
        s = lax.axis_index("s")
        wid = c * 16 + s
        pltpu.sync_copy(z_hbm, zbuf)
        pltpu.sync_copy(o_hbm, obuf)
        _zero_acc(s, zbuf, cacc)
        plsc.subcore_barrier()
        nch = E // CH
        nc = jnp.where(wid < nch % NT, nch // NT + 1, nch // NT)

        def body(t, carry):
            base = (wid + NT * t) * CH
            pltpu.sync_copy(dst_hbm.at[pl.ds(base, CH)], idx)
            pltpu.sync_copy(obuf, cacc.at[idx], add=True)
            return carry

        lax.fori_loop(0, nc, body, 0)
        plsc.subcore_barrier()
        _write_out(c, s, cacc, zbuf, cout_hbm)

    return k(dstv, z128, o128)


def _d1_0_body(h_ref, pa_ref, pb_ref, ca_ref, cb_ref, t_ref, su_ref, ic_ref):
    cnt = ca_ref[:, 0:1] + cb_ref[:, 0:1]
    ic = 1.0 / jnp.maximum(cnt, 1.0)
    ic_ref[...] = jnp.broadcast_to(ic, (BN_, 16))
    t = h_ref[...] + (pa_ref[...] + pb_ref[...]) * ic
    t_ref[...] = t
    s1 = jnp.sum(t, axis=0, keepdims=True)
    s2 = jnp.sum(t * t, axis=0, keepdims=True)
    su_ref[...] = jnp.concatenate([s1, s2, jnp.zeros((6, D), F32)], axis=0)[None]


def _d1_0(h, pout, cout):
    nb = N // BN_
    return pl.pallas_call(
        _d1_0_body,
        grid=(nb,),
        in_specs=[pl.BlockSpec((BN_, D), lambda j: (j, 0)),
                  pl.BlockSpec((BN_, D), lambda j: (j, 0)),
                  pl.BlockSpec((BN_, D), lambda j: (j + nb, 0)),
                  pl.BlockSpec((BN_, D), lambda j: (j, 0)),
                  pl.BlockSpec((BN_, D), lambda j: (j + nb, 0))],
        out_specs=[pl.BlockSpec((BN_, D), lambda j: (j, 0)),
                   pl.BlockSpec((1, 8, D), lambda j: (j, 0, 0)),
                   pl.BlockSpec((BN_, 16), lambda j: (j, 0))],
        out_shape=[_sds((N, D)), _sds((nb, 8, D)), _sds((N, 16))],
    )(h, pout, pout, cout, cout)


def _d1_body(h_ref, pa_ref, pb_ref, ic_ref, t_ref, su_ref):
    t = h_ref[...] + (pa_ref[...] + pb_ref[...]) * ic_ref[:, 0:1]
    t_ref[...] = t
    s1 = jnp.sum(t, axis=0, keepdims=True)
    s2 = jnp.sum(t * t, axis=0, keepdims=True)
    su_ref[...] = jnp.concatenate([s1, s2, jnp.zeros((6, D), F32)], axis=0)[None]


def _d1(h, pout, icnt):
    nb = N // BN_
    return pl.pallas_call(
        _d1_body,
        grid=(nb,),
        in_specs=[pl.BlockSpec((BN_, D), lambda j: (j, 0)),
                  pl.BlockSpec((BN_, D), lambda j: (j, 0)),
                  pl.BlockSpec((BN_, D), lambda j: (j + nb, 0)),
                  pl.BlockSpec((BN_, 16), lambda j: (j, 0))],
        out_specs=[pl.BlockSpec((BN_, D), lambda j: (j, 0)),
                   pl.BlockSpec((1, 8, D), lambda j: (j, 0, 0))],
        out_shape=[_sds((N, D)), _sds((nb, 8, D))],
    )(h, pout, pout, icnt)


def _d2_body(t_ref, su_ref, g_ref, b_ref, o_ref):
    su = su_ref[...]
    mu = jnp.sum(su[:, 0, :], axis=0) * (1.0 / N)
    ex2 = jnp.sum(su[:, 1, :], axis=0) * (1.0 / N)
    var = ex2 - mu * mu
    o_ref[...] = ((t_ref[...] - mu[None, :]) * lax.rsqrt(var + 1e-5)[None, :]
                  * g_ref[0:1] + b_ref[0:1])


def _d2(t, sums, g8, b8):
    nb = N // BN_
    return pl.pallas_call(
        _d2_body,
        grid=(nb,),
        in_specs=[pl.BlockSpec((BN_, D), lambda j: (j, 0)),
                  pl.BlockSpec((nb, 8, D), lambda j: (0, 0, 0)),
                  pl.BlockSpec((8, D), lambda j: (0, 0)),
                  pl.BlockSpec((8, D), lambda j: (0, 0))],
        out_specs=pl.BlockSpec((BN_, D), lambda j: (j, 0)),
        out_shape=_sds((N, D)),
    )(t, sums, g8, b8)


def kernel(x, pos, edge_index, emb_table, pre_W, pre_b, rbf_means, rbf_betas,
           Wf0, bf0, Ws0, bs0, gamma0, beta0,
           Wf1, bf1, Ws1, bs1, gamma1, beta1,
           Wf2, bf2, Ws2, bs2, gamma2, beta2):
    src = edge_index[0].astype(jnp.int32)
    dst = edge_index[1].astype(jnp.int32)
    x3 = x.astype(jnp.int32).reshape(N // BN_, 1, BN_)
    pos16 = jnp.pad(pos.astype(F32), ((0, 0), (0, 13)))
    emb_pad = jnp.pad(emb_table, ((0, D - emb_table.shape[0]), (0, 0)))
    bc8 = lambda v: jnp.broadcast_to(v[None, :], (8, v.shape[0]))
    z128 = jnp.zeros((CH, D), F32)
    o128 = jnp.ones((CH, D), F32)

    layers = []
    for (Wf, bf, Ws, bs, g, b) in ((Wf0, bf0, Ws0, bs0, gamma0, beta0),
                                   (Wf1, bf1, Ws1, bs1, gamma1, beta1),
                                   (Wf2, bf2, Ws2, bs2, gamma2, beta2)):
        WD = jnp.concatenate([Wf[:D], Ws[:D]], axis=1)
        WS = jnp.concatenate([Wf[D:2 * D], Ws[D:2 * D]], axis=1)
        WR = jnp.concatenate([Wf[2 * D:], Ws[2 * D:]], axis=1)
        bfs8 = bc8(jnp.concatenate([bf, bs]))
        layers.append((WD, WS, WR, bfs8, bc8(g), bc8(b)))

    h = _p1(x3, emb_pad, pre_W, bc8(pre_b))

    rbf = None
    icnt = None
    for i, (WD, WS, WR, bfs8, g8, b8) in enumerate(layers):
        if i == 0:
            t0 = jnp.concatenate([h, pos16, jnp.zeros((N, D - 16), F32)], axis=1)
            gd, gs = _sc_gather0(t0, dst, src)
            m, rbf = _tcb0(gd, gs, bc8(rbf_means), bc8(rbf_betas),
                           WD, WS, WR, bfs8)
            pout = _sc_scatter(m, dst, z128)
            cout = _sc_scatter_cnt(dst, z128, o128)
            t, sums, icnt = _d1_0(h, pout, cout)
        else:
            gd, gs = _sc_gather(h, dst, src)
            m = _tcb(gd, gs, rbf, WD, WS, WR, bfs8)
            pout = _sc_scatter(m, dst, z128)
            t, sums = _d1(h, pout, icnt)
        h = _d2(t, sums, g8, b8)
    return h


# speedup vs baseline: 1.4785x; 1.0633x over previous
import functools

import jax
import jax.numpy as jnp
from jax import lax
from jax.experimental import pallas as pl
from jax.experimental.pallas import tpu as pltpu
from jax.experimental.pallas import tpu_sc as plsc

N = 10000
D = 128
R = 32
E = 160000
BN_ = 1000
BE = 1000
CH = 128
CH0 = 64
NT = 32
NRC = 78
F32 = jnp.float32


def _mesh():
    return plsc.VectorSubcoreMesh(core_axis_name="c", subcore_axis_name="s")


def _sds(shape):
    return jax.ShapeDtypeStruct(shape, F32)


def _p1_body(x_ref, e_ref, w_ref, b_ref, o_ref):
    xt = x_ref[0, 0]
    oh = (xt[:, None] == lax.broadcasted_iota(jnp.int32, (BN_, D), 1)).astype(F32)
    emb = jnp.dot(oh, e_ref[...], preferred_element_type=F32)
    o_ref[...] = jnp.maximum(jnp.dot(emb, w_ref[...], preferred_element_type=F32)
                             + b_ref[0:1], 0.0)


def _p1(x3, emb_pad, pre_W, pre_b8):
    return pl.pallas_call(
        _p1_body,
        grid=(N // BN_,),
        in_specs=[pl.BlockSpec((1, 1, BN_), lambda j: (j, 0, 0)),
                  pl.BlockSpec((D, D), lambda j: (0, 0)),
                  pl.BlockSpec((D, D), lambda j: (0, 0)),
                  pl.BlockSpec((8, D), lambda j: (0, 0))],
        out_specs=pl.BlockSpec((BN_, D), lambda j: (j, 0)),
        out_shape=_sds((N, D)),
    )(x3, emb_pad, pre_W, pre_b8)


def _mk_gather(width, ch):
    nch = E // ch

    @functools.partial(
        pl.kernel, mesh=_mesh(),
        out_type=[_sds((E, width)), _sds((E, width))],
        scratch_types=[pltpu.VMEM((ch,), jnp.int32), pltpu.VMEM((ch,), jnp.int32),
                       pltpu.VMEM((ch, width), F32), pltpu.VMEM((ch, width), F32),
                       pltpu.SemaphoreType.DMA, pltpu.SemaphoreType.DMA],
    )
    def k(tab_hbm, dst_hbm, src_hbm, gd_hbm, gs_hbm,
          ixd, ixs, rd, rs, sd, ss):
        wid = lax.axis_index("c") * 16 + lax.axis_index("s")
        nc = jnp.where(wid < nch % NT, nch // NT + 1, nch // NT)

        def body(t, carry):
            base = (wid + NT * t) * ch
            pltpu.sync_copy(dst_hbm.at[pl.ds(base, ch)], ixd)
            pltpu.sync_copy(src_hbm.at[pl.ds(base, ch)], ixs)
            c0 = pltpu.async_copy(tab_hbm.at[ixd], rd, sd)
            c1 = pltpu.async_copy(tab_hbm.at[ixs], rs, ss)
            c0.wait(); c1.wait()
            pltpu.sync_copy(rd, gd_hbm.at[pl.ds(base, ch)])
            pltpu.sync_copy(rs, gs_hbm.at[pl.ds(base, ch)])
            return carry

        lax.fori_loop(0, nc, body, 0)

    return k


def _sc_gather0(t0, dstv, srcv):
    return _mk_gather(2 * D, CH)(t0, dstv, srcv)


def _sc_gather(h, dstv, srcv):
    return _mk_gather(D, CH)(h, dstv, srcv)


def _edge_math(gd, gs, rbf, wd_ref, ws_ref, wr_ref, bfs_ref):
    z = (jnp.dot(gd, wd_ref[...], preferred_element_type=F32)
         + jnp.dot(gs, ws_ref[...], preferred_element_type=F32)
         + jnp.dot(rbf, wr_ref[...], preferred_element_type=F32)
         + bfs_ref[0:1])
    return jax.nn.sigmoid(z[:, :D]) * jax.nn.softplus(z[:, D:])


def _tcb0_body(gd_ref, gs_ref, mn_ref, bt_ref,
               wd_ref, ws_ref, wr_ref, bfs_ref, m_ref, rbf_ref):
    gd = gd_ref[:, :D]
    gs = gs_ref[:, :D]
    vec = gs_ref[:, D:D + 16] - gd_ref[:, D:D + 16]
    d = jnp.sqrt(jnp.sum(vec * vec, axis=1, keepdims=True) + 1e-12)
    cut = jnp.where(d < 5.0, 0.5 * (jnp.cos(d * (jnp.pi / 5.0)) + 1.0), 0.0)
    rbf = cut * jnp.exp(-bt_ref[0:1] * (jnp.exp(-d) - mn_ref[0:1]) ** 2)
    rbf_ref[...] = rbf
    m_ref[...] = _edge_math(gd, gs, rbf, wd_ref, ws_ref, wr_ref, bfs_ref)


def _tcb0(gd, gs, means8, betas8, WD, WS, WR, bfs8):
    full = lambda s: pl.BlockSpec(s, lambda j: tuple(0 for _ in s))
    return pl.pallas_call(
        _tcb0_body,
        grid=(E // BE,),
        in_specs=[pl.BlockSpec((BE, 2 * D), lambda j: (j, 0)),
                  pl.BlockSpec((BE, 2 * D), lambda j: (j, 0)),
                  full((8, R)), full((8, R)),
                  full((D, 2 * D)), full((D, 2 * D)), full((R, 2 * D)),
                  full((8, 2 * D))],
        out_specs=[pl.BlockSpec((BE, D), lambda j: (j, 0)),
                   pl.BlockSpec((BE, R), lambda j: (j, 0))],
        out_shape=[_sds((E, D)), _sds((E, R))],
    )(gd, gs, means8, betas8, WD, WS, WR, bfs8)


def _tcb_body(gd_ref, gs_ref, rbf_ref, wd_ref, ws_ref, wr_ref, bfs_ref, m_ref):
    m_ref[...] = _edge_math(gd_ref[...], gs_ref[...], rbf_ref[...],
                            wd_ref, ws_ref, wr_ref, bfs_ref)


def _tcb(gd, gs, rbf, WD, WS, WR, bfs8):
    full = lambda s: pl.BlockSpec(s, lambda j: tuple(0 for _ in s))
    return pl.pallas_call(
        _tcb_body,
        grid=(E // BE,),
        in_specs=[pl.BlockSpec((BE, D), lambda j: (j, 0)),
                  pl.BlockSpec((BE, D), lambda j: (j, 0)),
                  pl.BlockSpec((BE, R), lambda j: (j, 0)),
                  full((D, 2 * D)), full((D, 2 * D)), full((R, 2 * D)),
                  full((8, 2 * D))],
        out_specs=pl.BlockSpec((BE, D), lambda j: (j, 0)),
        out_shape=_sds((E, D)),
    )(gd, gs, rbf, WD, WS, WR, bfs8)


def _zero_acc(s, zbuf, acc):
    for t in range(5):
        ci = s + 16 * t

        @pl.when(ci < NRC)
        def _():
            pltpu.sync_copy(zbuf, acc.at[pl.ds(ci * CH, CH)])

        @pl.when(ci == NRC)
        def _():
            pltpu.sync_copy(zbuf.at[pl.ds(0, 16)], acc.at[pl.ds(NRC * CH, 16)])


def _write_out(c, s, acc, buf, out_hbm):
    for t in range(5):
        ci = s + 16 * t

        @pl.when(ci < NRC)
        def _():
            pltpu.sync_copy(acc.at[pl.ds(ci * CH, CH)], buf)
            pltpu.sync_copy(buf, out_hbm.at[pl.ds(c * N + ci * CH, CH)])

        @pl.when(ci == NRC)
        def _():
            pltpu.sync_copy(acc.at[pl.ds(NRC * CH, 16)], buf.at[pl.ds(0, 16)])
            pltpu.sync_copy(buf.at[pl.ds(0, 16)],
                            out_hbm.at[pl.ds(c * N + NRC * CH, 16)])


def _sc_scatter(m, dstv, z128):
    @functools.partial(
        pl.kernel, mesh=_mesh(),
        out_type=_sds((2 * N, D)),
        scratch_types=[pltpu.VMEM((CH, D), F32), pltpu.VMEM((CH, D), F32),
                       pltpu.VMEM((CH,), jnp.int32), pltpu.VMEM((CH,), jnp.int32),
                       pltpu.VMEM_SHARED((N, D), F32),
                       pltpu.SemaphoreType.DMA, pltpu.SemaphoreType.DMA,
                       pltpu.SemaphoreType.DMA, pltpu.SemaphoreType.DMA],
    )
    def k(m_hbm, dst_hbm, z_hbm, pout_hbm, mb0, mb1, ix0, ix1, acc,
          sm0, sm1, si0, si1):
        c = lax.axis_index("c")
        s = lax.axis_index("s")
        wid = c * 16 + s
        pltpu.sync_copy(z_hbm, mb0)
        _zero_acc(s, mb0, acc)
        plsc.subcore_barrier()
        nch = E // CH
        nc = jnp.where(wid < nch % NT, nch // NT + 1, nch // NT)
        kmax = (nch // NT + 1) // 2

        def base(t):
            return (wid + NT * t) * CH

        pltpu.async_copy(dst_hbm.at[pl.ds(base(0), CH)], ix0, si0)
        pltpu.async_copy(m_hbm.at[pl.ds(base(0), CH)], mb0, sm0)

        def body(kk, carry):
            t0 = 2 * kk
            b0 = base(t0)
            b1 = base(t0 + 1)

            @pl.when(t0 + 1 < nc)
            def _():
                pltpu.async_copy(dst_hbm.at[pl.ds(b1, CH)], ix1, si1)
                pltpu.async_copy(m_hbm.at[pl.ds(b1, CH)], mb1, sm1)

            pltpu.make_async_copy(dst_hbm.at[pl.ds(b0, CH)], ix0, si0).wait()
            pltpu.make_async_copy(m_hbm.at[pl.ds(b0, CH)], mb0, sm0).wait()
            pltpu.sync_copy(mb0, acc.at[ix0], add=True)

            @pl.when(kk < kmax - 1)
            def _():
                b2 = base(t0 + 2)
                pltpu.async_copy(dst_hbm.at[pl.ds(b2, CH)], ix0, si0)
                pltpu.async_copy(m_hbm.at[pl.ds(b2, CH)], mb0, sm0)

            @pl.when(t0 + 1 < nc)
            def _():
                pltpu.make_async_copy(dst_hbm.at[pl.ds(b1, CH)], ix1, si1).wait()
                pltpu.make_async_copy(m_hbm.at[pl.ds(b1, CH)], mb1, sm1).wait()
                pltpu.sync_copy(mb1, acc.at[ix1], add=True)
            return carry

        lax.fori_loop(0, kmax, body, 0)
        plsc.subcore_barrier()
        _write_out(c, s, acc, mb0, pout_hbm)

    return k(m, dstv, z128)


def _sc_scatter_cnt(dstv, z128, o128):
    @functools.partial(
        pl.kernel, mesh=_mesh(),
        out_type=_sds((2 * N, D)),
        scratch_types=[pltpu.VMEM((CH, D), F32), pltpu.VMEM((CH, D), F32),
                       pltpu.VMEM((CH,), jnp.int32),
                       pltpu.VMEM_SHARED((N, D), F32)],
    )
    def k(dst_hbm, z_hbm, o_hbm, cout_hbm, zbuf, obuf, idx, cacc):
        c = lax.axis_index("c")
        s = lax.axis_index("s")
        wid = c * 16 + s
        pltpu.sync_copy(z_hbm, zbuf)
        pltpu.sync_copy(o_hbm, obuf)
        _zero_acc(s, zbuf, cacc)
        plsc.subcore_barrier()
        nch = E // CH
        nc = jnp.where(wid < nch % NT, nch // NT + 1, nch // NT)

        def body(t, carry):
            base = (wid + NT * t) * CH
            pltpu.sync_copy(dst_hbm.at[pl.ds(base, CH)], idx)
            pltpu.sync_copy(obuf, cacc.at[idx], add=True)
            return carry

        lax.fori_loop(0, nc, body, 0)
        plsc.subcore_barrier()
        _write_out(c, s, cacc, zbuf, cout_hbm)

    return k(dstv, z128, o128)


def _d1_0_body(h_ref, pa_ref, pb_ref, ca_ref, cb_ref, t_ref, su_ref, ic_ref):
    cnt = ca_ref[:, 0:1] + cb_ref[:, 0:1]
    ic = 1.0 / jnp.maximum(cnt, 1.0)
    ic_ref[...] = jnp.broadcast_to(ic, (BN_, 16))
    t = h_ref[...] + (pa_ref[...] + pb_ref[...]) * ic
    t_ref[...] = t
    s1 = jnp.sum(t, axis=0, keepdims=True)
    s2 = jnp.sum(t * t, axis=0, keepdims=True)
    su_ref[...] = jnp.concatenate([s1, s2, jnp.zeros((6, D), F32)], axis=0)[None]


def _d1_0(h, pout, cout):
    nb = N // BN_
    return pl.pallas_call(
        _d1_0_body,
        grid=(nb,),
        in_specs=[pl.BlockSpec((BN_, D), lambda j: (j, 0)),
                  pl.BlockSpec((BN_, D), lambda j: (j, 0)),
                  pl.BlockSpec((BN_, D), lambda j: (j + nb, 0)),
                  pl.BlockSpec((BN_, D), lambda j: (j, 0)),
                  pl.BlockSpec((BN_, D), lambda j: (j + nb, 0))],
        out_specs=[pl.BlockSpec((BN_, D), lambda j: (j, 0)),
                   pl.BlockSpec((1, 8, D), lambda j: (j, 0, 0)),
                   pl.BlockSpec((BN_, 16), lambda j: (j, 0))],
        out_shape=[_sds((N, D)), _sds((nb, 8, D)), _sds((N, 16))],
    )(h, pout, pout, cout, cout)


def _d1_body(h_ref, pa_ref, pb_ref, ic_ref, t_ref, su_ref):
    t = h_ref[...] + (pa_ref[...] + pb_ref[...]) * ic_ref[:, 0:1]
    t_ref[...] = t
    s1 = jnp.sum(t, axis=0, keepdims=True)
    s2 = jnp.sum(t * t, axis=0, keepdims=True)
    su_ref[...] = jnp.concatenate([s1, s2, jnp.zeros((6, D), F32)], axis=0)[None]


def _d1(h, pout, icnt):
    nb = N // BN_
    return pl.pallas_call(
        _d1_body,
        grid=(nb,),
        in_specs=[pl.BlockSpec((BN_, D), lambda j: (j, 0)),
                  pl.BlockSpec((BN_, D), lambda j: (j, 0)),
                  pl.BlockSpec((BN_, D), lambda j: (j + nb, 0)),
                  pl.BlockSpec((BN_, 16), lambda j: (j, 0))],
        out_specs=[pl.BlockSpec((BN_, D), lambda j: (j, 0)),
                   pl.BlockSpec((1, 8, D), lambda j: (j, 0, 0))],
        out_shape=[_sds((N, D)), _sds((nb, 8, D))],
    )(h, pout, pout, icnt)


def _d2_body(t_ref, su_ref, g_ref, b_ref, o_ref):
    su = su_ref[...]
    mu = jnp.sum(su[:, 0, :], axis=0) * (1.0 / N)
    ex2 = jnp.sum(su[:, 1, :], axis=0) * (1.0 / N)
    var = ex2 - mu * mu
    o_ref[...] = ((t_ref[...] - mu[None, :]) * lax.rsqrt(var + 1e-5)[None, :]
                  * g_ref[0:1] + b_ref[0:1])


def _d2(t, sums, g8, b8):
    nb = N // BN_
    return pl.pallas_call(
        _d2_body,
        grid=(nb,),
        in_specs=[pl.BlockSpec((BN_, D), lambda j: (j, 0)),
                  pl.BlockSpec((nb, 8, D), lambda j: (0, 0, 0)),
                  pl.BlockSpec((8, D), lambda j: (0, 0)),
                  pl.BlockSpec((8, D), lambda j: (0, 0))],
        out_specs=pl.BlockSpec((BN_, D), lambda j: (j, 0)),
        out_shape=_sds((N, D)),
    )(t, sums, g8, b8)


def kernel(x, pos, edge_index, emb_table, pre_W, pre_b, rbf_means, rbf_betas,
           Wf0, bf0, Ws0, bs0, gamma0, beta0,
           Wf1, bf1, Ws1, bs1, gamma1, beta1,
           Wf2, bf2, Ws2, bs2, gamma2, beta2):
    src = edge_index[0].astype(jnp.int32)
    dst = edge_index[1].astype(jnp.int32)
    x3 = x.astype(jnp.int32).reshape(N // BN_, 1, BN_)
    pos16 = jnp.pad(pos.astype(F32), ((0, 0), (0, 13)))
    emb_pad = jnp.pad(emb_table, ((0, D - emb_table.shape[0]), (0, 0)))
    bc8 = lambda v: jnp.broadcast_to(v[None, :], (8, v.shape[0]))
    z128 = jnp.zeros((CH, D), F32)
    o128 = jnp.ones((CH, D), F32)

    layers = []
    for (Wf, bf, Ws, bs, g, b) in ((Wf0, bf0, Ws0, bs0, gamma0, beta0),
                                   (Wf1, bf1, Ws1, bs1, gamma1, beta1),
                                   (Wf2, bf2, Ws2, bs2, gamma2, beta2)):
        WD = jnp.concatenate([Wf[:D], Ws[:D]], axis=1)
        WS = jnp.concatenate([Wf[D:2 * D], Ws[D:2 * D]], axis=1)
        WR = jnp.concatenate([Wf[2 * D:], Ws[2 * D:]], axis=1)
        bfs8 = bc8(jnp.concatenate([bf, bs]))
        layers.append((WD, WS, WR, bfs8, bc8(g), bc8(b)))

    h = _p1(x3, emb_pad, pre_W, bc8(pre_b))

    rbf = None
    icnt = None
    for i, (WD, WS, WR, bfs8, g8, b8) in enumerate(layers):
        if i == 0:
            t0 = jnp.concatenate([h, pos16, jnp.zeros((N, D - 16), F32)], axis=1)
            gd, gs = _sc_gather0(t0, dst, src)
            m, rbf = _tcb0(gd, gs, bc8(rbf_means), bc8(rbf_betas),
                           WD, WS, WR, bfs8)
            pout = _sc_scatter(m, dst, z128)
            cout = _sc_scatter_cnt(dst, z128, o128)
            t, sums, icnt = _d1_0(h, pout, cout)
        else:
            gd, gs = _sc_gather(h, dst, src)
            m = _tcb(gd, gs, rbf, WD, WS, WR, bfs8)
            pout = _sc_scatter(m, dst, z128)
            t, sums = _d1(h, pout, icnt)
        h = _d2(t, sums, g8, b8)
    return h
